# Initial kernel scaffold; baseline (speedup 1.0000x reference)
import jax
import jax.numpy as jnp
from jax.experimental import pallas as pl


def kernel(x, W, scale, per_expert_scale):
    raise NotImplementedError("write your pallas kernel here")


# speedup vs baseline: 1.2340x; 1.2340x over previous
import functools

import jax
import jax.numpy as jnp
from jax import lax
from jax.experimental import pallas as pl
from jax.experimental.pallas import tpu as pltpu
from jax.experimental.pallas import tpu_sc as plsc

HIDDEN = 4096
E = 64
TOK = 16384
EPS = 1e-06

TB = 256
NBLK = TOK // TB
NC, NS = 2, 16
NW = NC * NS
CHUNK = TOK // NW


def _router_block(x_ref, wt_ref, scale_ref, pes_ref,
                  s0_ref, s1_ref, e0_ref, e1_ref, r0_ref, r1_ref, cnt_ref,
                  run_ref):
    i = pl.program_id(0)

    @pl.when(i == 0)
    def _():
        run_ref[...] = jnp.zeros_like(run_ref)

    xf = x_ref[...]
    ms = jnp.mean(xf * xf, axis=-1, keepdims=True)
    normed = (xf * lax.rsqrt(ms + EPS)) * scale_ref[...]
    logits = jnp.dot(normed, wt_ref[...], preferred_element_type=jnp.float32)
    scores = jax.nn.sigmoid(logits * pes_ref[...])

    iota = lax.broadcasted_iota(jnp.int32, (TB, E), 1)
    m1 = jnp.max(scores, axis=1, keepdims=True)
    i1 = jnp.min(jnp.where(scores == m1, iota, E), axis=1, keepdims=True)
    masked = jnp.where(iota == i1, -1.0, scores)
    m2 = jnp.max(masked, axis=1, keepdims=True)
    i2 = jnp.min(jnp.where(masked == m2, iota, E), axis=1, keepdims=True)

    oh0 = (iota == i1).astype(jnp.float32)
    oh1 = (iota == i2).astype(jnp.float32)
    ohs = oh0 + oh1

    ri = lax.broadcasted_iota(jnp.int32, (TB, TB), 0)
    ci = lax.broadcasted_iota(jnp.int32, (TB, TB), 1)
    ltri = (ci < ri).astype(jnp.float32)
    S = jnp.dot(ltri, ohs, preferred_element_type=jnp.float32)

    srun = S + run_ref[...]
    r0 = jnp.sum(srun * oh0, axis=1, keepdims=True)
    r1 = jnp.sum(srun * oh1, axis=1, keepdims=True)

    s0_ref[...] = m1
    s1_ref[...] = m2
    e0_ref[...] = i1
    e1_ref[...] = i2
    r0_ref[...] = r0.astype(jnp.int32)
    r1_ref[...] = r1.astype(jnp.int32)

    run_new = run_ref[...] + jnp.sum(ohs, axis=0, keepdims=True)
    run_ref[...] = run_new
    cnt_ref[...] = run_new.astype(jnp.int32)


def _stage1(x, wt, scale2d, pes2d):
    grid = (NBLK,)
    col = lambda dt: jax.ShapeDtypeStruct((TOK, 1), dt)
    out_shape = (col(jnp.float32), col(jnp.float32),
                 col(jnp.int32), col(jnp.int32),
                 col(jnp.int32), col(jnp.int32),
                 jax.ShapeDtypeStruct((1, E), jnp.int32))
    colspec = pl.BlockSpec((TB, 1), lambda i: (i, 0))
    return pl.pallas_call(
        _router_block,
        grid=grid,
        in_specs=[
            pl.BlockSpec((TB, HIDDEN), lambda i: (i, 0)),
            pl.BlockSpec((HIDDEN, E), lambda i: (0, 0)),
            pl.BlockSpec((1, HIDDEN), lambda i: (0, 0)),
            pl.BlockSpec((1, E), lambda i: (0, 0)),
        ],
        out_specs=(colspec, colspec, colspec, colspec, colspec, colspec,
                   pl.BlockSpec((1, E), lambda i: (0, 0))),
        out_shape=out_shape,
        scratch_shapes=[pltpu.VMEM((1, E), jnp.float32)],
    )(x, wt, scale2d, pes2d)


def _scatter_body(s0, s1, e0, e1, r0, r1, cnt, out_s, out_t,
                  cnt_v, off_v, e_v, r_v, val_v, tok_v, dest_v, sem):
    wid = lax.axis_index("s") * NC + lax.axis_index("c")
    base = wid * CHUNK

    pltpu.sync_copy(cnt, cnt_v)

    carry = jnp.int32(0)
    for j in range(E // 16):
        v = cnt_v[pl.ds(j * 16, 16)]
        c = plsc.cumsum(v)
        off_v[pl.ds(j * 16, 16)] = (c - v) + carry
        carry = carry + jnp.sum(v)

    for k in range(CHUNK // 16):
        tok_v[pl.ds(k * 16, 16)] = lax.iota(jnp.int32, 16) + (base + k * 16)

    for col in range(2):
        e_hbm = (e0, e1)[col]
        r_hbm = (r0, r1)[col]
        s_hbm = (s0, s1)[col]
        pltpu.sync_copy(e_hbm.at[pl.ds(base, CHUNK)], e_v)
        pltpu.sync_copy(r_hbm.at[pl.ds(base, CHUNK)], r_v)
        pltpu.sync_copy(s_hbm.at[pl.ds(base, CHUNK)], val_v)
        for j in range(CHUNK // 128):
            for i in range(8):
                k = j * 8 + i
                ev = e_v[pl.ds(k * 16, 16)]
                rv = r_v[pl.ds(k * 16, 16)]
                dest_v[j, pl.ds(i * 16, 16)] = plsc.load_gather(off_v, [ev]) + rv
        copies = []
        for j in range(CHUNK // 128):
            src = pl.ds(j * 128, 128)
            copies.append(pltpu.async_copy(val_v.at[src], out_s.at[dest_v.at[j]], sem))
            copies.append(pltpu.async_copy(tok_v.at[src], out_t.at[dest_v.at[j]], sem))
        for cp in copies:
            cp.wait()


def _stage2(s0, s1, e0, e1, r0, r1, cnt):
    mesh = plsc.VectorSubcoreMesh(core_axis_name="c", subcore_axis_name="s",
                                  num_cores=NC, num_subcores=NS)
    run = pl.kernel(
        _scatter_body,
        out_type=(jax.ShapeDtypeStruct((2 * TOK,), jnp.float32),
                  jax.ShapeDtypeStruct((2 * TOK,), jnp.int32)),
        mesh=mesh,
        scratch_types=[
            pltpu.VMEM((E,), jnp.int32),
            pltpu.VMEM((E,), jnp.int32),
            pltpu.VMEM((CHUNK,), jnp.int32),
            pltpu.VMEM((CHUNK,), jnp.int32),
            pltpu.VMEM((CHUNK,), jnp.float32),
            pltpu.VMEM((CHUNK,), jnp.int32),
            pltpu.VMEM((CHUNK // 128, 128), jnp.int32),
            pltpu.SemaphoreType.DMA,
        ],
        compiler_params=pltpu.CompilerParams(needs_layout_passes=False),
    )
    return run(s0, s1, e0, e1, r0, r1, cnt)


def kernel(x, W, scale, per_expert_scale):
    wt = W.T
    scale2d = scale.reshape(1, HIDDEN)
    pes2d = per_expert_scale.reshape(1, E)
    s0, s1, e0, e1, r0, r1, cnt = _stage1(x, wt, scale2d, pes2d)
    out_s, out_t = _stage2(s0.reshape(TOK), s1.reshape(TOK),
                           e0.reshape(TOK), e1.reshape(TOK),
                           r0.reshape(TOK), r1.reshape(TOK),
                           cnt.reshape(E))
    return out_s, out_t, cnt.reshape(E)


# speedup vs baseline: 1.8119x; 1.4683x over previous
import functools

import jax
import jax.numpy as jnp
from jax import lax
from jax.experimental import pallas as pl
from jax.experimental.pallas import tpu as pltpu
from jax.experimental.pallas import tpu_sc as plsc

HIDDEN = 4096
E = 64
TOK = 16384
EPS = 1e-06

TB = 256
NBLK = TOK // TB
NC, NS = 2, 16
SCHUNK = TOK // NS


def _router_block(x_ref, wt_ref, scale_ref, pes_ref,
                  s0_ref, s1_ref, e0_ref, e1_ref, r0_ref, r1_ref, cnt_ref,
                  run_ref):
    i = pl.program_id(0)

    @pl.when(i == 0)
    def _():
        run_ref[...] = jnp.zeros_like(run_ref)

    xf = x_ref[...]
    ms = jnp.mean(xf * xf, axis=-1, keepdims=True)
    normed = (xf * lax.rsqrt(ms + EPS)) * scale_ref[...]
    logits = jnp.dot(normed, wt_ref[...], preferred_element_type=jnp.float32)
    scores = jax.nn.sigmoid(logits * pes_ref[...])

    iota = lax.broadcasted_iota(jnp.int32, (TB, E), 1)
    m1 = jnp.max(scores, axis=1, keepdims=True)
    i1 = jnp.min(jnp.where(scores == m1, iota, E), axis=1, keepdims=True)
    masked = jnp.where(iota == i1, -1.0, scores)
    m2 = jnp.max(masked, axis=1, keepdims=True)
    i2 = jnp.min(jnp.where(masked == m2, iota, E), axis=1, keepdims=True)

    oh0 = (iota == i1).astype(jnp.float32)
    oh1 = (iota == i2).astype(jnp.float32)
    ohs = oh0 + oh1

    ri = lax.broadcasted_iota(jnp.int32, (TB, TB), 0)
    ci = lax.broadcasted_iota(jnp.int32, (TB, TB), 1)
    ltri = (ci < ri).astype(jnp.float32)
    S = jnp.dot(ltri, ohs, preferred_element_type=jnp.float32)

    srun = S + run_ref[...]
    r0 = jnp.sum(srun * oh0, axis=1, keepdims=True)
    r1 = jnp.sum(srun * oh1, axis=1, keepdims=True)

    s0_ref[...] = m1
    s1_ref[...] = m2
    e0_ref[...] = i1
    e1_ref[...] = i2
    r0_ref[...] = r0.astype(jnp.int32)
    r1_ref[...] = r1.astype(jnp.int32)

    run_new = run_ref[...] + jnp.sum(ohs, axis=0, keepdims=True)
    run_ref[...] = run_new
    cnt_ref[...] = run_new.astype(jnp.int32)


def _stage1(x, wt, scale2d, pes2d):
    grid = (NBLK,)
    col = lambda dt: jax.ShapeDtypeStruct((TOK, 1), dt)
    out_shape = (col(jnp.float32), col(jnp.float32),
                 col(jnp.int32), col(jnp.int32),
                 col(jnp.int32), col(jnp.int32),
                 jax.ShapeDtypeStruct((1, E), jnp.int32))
    colspec = pl.BlockSpec((TB, 1), lambda i: (i, 0))
    return pl.pallas_call(
        _router_block,
        grid=grid,
        in_specs=[
            pl.BlockSpec((TB, HIDDEN), lambda i: (i, 0)),
            pl.BlockSpec((HIDDEN, E), lambda i: (0, 0)),
            pl.BlockSpec((1, HIDDEN), lambda i: (0, 0)),
            pl.BlockSpec((1, E), lambda i: (0, 0)),
        ],
        out_specs=(colspec, colspec, colspec, colspec, colspec, colspec,
                   pl.BlockSpec((1, E), lambda i: (0, 0))),
        out_shape=out_shape,
        scratch_shapes=[pltpu.VMEM((1, E), jnp.float32)],
    )(x, wt, scale2d, pes2d)


def _scatter_body(s0, s1, e0, e1, r0, r1, cnt, out_s, out_t,
                  sh_s, sh_t, cnt_v, off_v, e_v, r_v, val_v, tok_v, dest_v,
                  stage_s, stage_t, sem):
    wid = lax.axis_index("s")
    base = wid * SCHUNK

    pltpu.sync_copy(cnt, cnt_v)

    carry = jnp.int32(0)
    for j in range(E // 16):
        v = cnt_v[pl.ds(j * 16, 16)]
        c = plsc.cumsum(v)
        off_v[pl.ds(j * 16, 16)] = (c - v) + carry
        carry = carry + jnp.sum(v)

    for k in range(SCHUNK // 16):
        tok_v[pl.ds(k * 16, 16)] = lax.iota(jnp.int32, 16) + (base + k * 16)

    copies = []
    for col in range(2):
        e_hbm = (e0, e1)[col]
        r_hbm = (r0, r1)[col]
        s_hbm = (s0, s1)[col]
        pltpu.sync_copy(e_hbm.at[pl.ds(base, SCHUNK)], e_v)
        pltpu.sync_copy(r_hbm.at[pl.ds(base, SCHUNK)], r_v)
        pltpu.sync_copy(s_hbm.at[pl.ds(base, SCHUNK)], val_v)
        for j in range(SCHUNK // 128):
            for i in range(8):
                k = j * 8 + i
                ev = e_v[pl.ds(k * 16, 16)]
                rv = r_v[pl.ds(k * 16, 16)]
                dest_v[col, j, pl.ds(i * 16, 16)] = plsc.load_gather(off_v, [ev]) + rv
        for j in range(SCHUNK // 128):
            src = pl.ds(j * 128, 128)
            copies.append(pltpu.async_copy(val_v.at[src], sh_s.at[dest_v.at[col, j]], sem))
            copies.append(pltpu.async_copy(tok_v.at[src], sh_t.at[dest_v.at[col, j]], sem))
        if col == 0:
            for cp in copies:
                cp.wait()
            copies = []
    for cp in copies:
        cp.wait()
    plsc.subcore_barrier()
    out_slice = pl.ds(wid * (2 * TOK // NS), 2 * TOK // NS)
    pltpu.sync_copy(sh_s.at[out_slice], stage_s)
    pltpu.sync_copy(stage_s, out_s.at[out_slice])
    pltpu.sync_copy(sh_t.at[out_slice], stage_t)
    pltpu.sync_copy(stage_t, out_t.at[out_slice])


def _stage2(s0, s1, e0, e1, r0, r1, cnt):
    mesh = plsc.VectorSubcoreMesh(core_axis_name="c", subcore_axis_name="s",
                                  num_cores=1, num_subcores=NS)
    run = pl.kernel(
        _scatter_body,
        out_type=(jax.ShapeDtypeStruct((2 * TOK,), jnp.float32),
                  jax.ShapeDtypeStruct((2 * TOK,), jnp.int32)),
        mesh=mesh,
        scratch_types=[
            pltpu.VMEM_SHARED((2 * TOK,), jnp.float32),
            pltpu.VMEM_SHARED((2 * TOK,), jnp.int32),
            pltpu.VMEM((E,), jnp.int32),
            pltpu.VMEM((E,), jnp.int32),
            pltpu.VMEM((SCHUNK,), jnp.int32),
            pltpu.VMEM((SCHUNK,), jnp.int32),
            pltpu.VMEM((SCHUNK,), jnp.float32),
            pltpu.VMEM((SCHUNK,), jnp.int32),
            pltpu.VMEM((2, SCHUNK // 128, 128), jnp.int32),
            pltpu.VMEM((2 * TOK // NS,), jnp.float32),
            pltpu.VMEM((2 * TOK // NS,), jnp.int32),
            pltpu.SemaphoreType.DMA,
        ],
        compiler_params=pltpu.CompilerParams(needs_layout_passes=False),
    )
    return run(s0, s1, e0, e1, r0, r1, cnt)


def kernel(x, W, scale, per_expert_scale):
    wt = W.T
    scale2d = scale.reshape(1, HIDDEN)
    pes2d = per_expert_scale.reshape(1, E)
    s0, s1, e0, e1, r0, r1, cnt = _stage1(x, wt, scale2d, pes2d)
    out_s, out_t = _stage2(s0.reshape(TOK), s1.reshape(TOK),
                           e0.reshape(TOK), e1.reshape(TOK),
                           r0.reshape(TOK), r1.reshape(TOK),
                           cnt.reshape(E))
    return out_s, out_t, cnt.reshape(E)


# speedup vs baseline: 2.0669x; 1.1407x over previous
import functools

import jax
import jax.numpy as jnp
from jax import lax
from jax.experimental import pallas as pl
from jax.experimental.pallas import tpu as pltpu
from jax.experimental.pallas import tpu_sc as plsc

HIDDEN = 4096
E = 64
TOK = 16384
EPS = 1e-06

TB = 512
NBLK = TOK // TB
NC, NS = 2, 16
SCHUNK = TOK // NS


def _router_block(x_ref, wt_ref, scale_ref, pes_ref,
                  s0_ref, s1_ref, e0_ref, e1_ref, r0_ref, r1_ref, cnt_ref,
                  run_ref):
    i = pl.program_id(0)

    @pl.when(i == 0)
    def _():
        run_ref[...] = jnp.zeros_like(run_ref)

    xf = x_ref[...]
    ms = jnp.mean(xf * xf, axis=-1, keepdims=True)
    normed = (xf * lax.rsqrt(ms + EPS)) * scale_ref[...]
    logits = jnp.dot(normed, wt_ref[...], preferred_element_type=jnp.float32)
    scores = jax.nn.sigmoid(logits * pes_ref[...])

    iota = lax.broadcasted_iota(jnp.int32, (TB, E), 1)
    m1 = jnp.max(scores, axis=1, keepdims=True)
    i1 = jnp.min(jnp.where(scores == m1, iota, E), axis=1, keepdims=True)
    masked = jnp.where(iota == i1, -1.0, scores)
    m2 = jnp.max(masked, axis=1, keepdims=True)
    i2 = jnp.min(jnp.where(masked == m2, iota, E), axis=1, keepdims=True)

    oh0 = (iota == i1).astype(jnp.float32)
    oh1 = (iota == i2).astype(jnp.float32)
    ohs = oh0 + oh1

    ri = lax.broadcasted_iota(jnp.int32, (TB, TB), 0)
    ci = lax.broadcasted_iota(jnp.int32, (TB, TB), 1)
    ltri = (ci < ri).astype(jnp.float32)
    S = jnp.dot(ltri, ohs, preferred_element_type=jnp.float32)

    srun = S + run_ref[...]
    r0 = jnp.sum(srun * oh0, axis=1, keepdims=True)
    r1 = jnp.sum(srun * oh1, axis=1, keepdims=True)

    s0_ref[...] = m1
    s1_ref[...] = m2
    e0_ref[...] = i1
    e1_ref[...] = i2
    r0_ref[...] = r0.astype(jnp.int32)
    r1_ref[...] = r1.astype(jnp.int32)

    run_new = run_ref[...] + jnp.sum(ohs, axis=0, keepdims=True)
    run_ref[...] = run_new
    cnt_ref[...] = run_new.astype(jnp.int32)


def _stage1(x, wt, scale2d, pes2d):
    grid = (NBLK,)
    col = lambda dt: jax.ShapeDtypeStruct((TOK, 1), dt)
    out_shape = (col(jnp.float32), col(jnp.float32),
                 col(jnp.int32), col(jnp.int32),
                 col(jnp.int32), col(jnp.int32),
                 jax.ShapeDtypeStruct((1, E), jnp.int32))
    colspec = pl.BlockSpec((TB, 1), lambda i: (i, 0))
    return pl.pallas_call(
        _router_block,
        grid=grid,
        in_specs=[
            pl.BlockSpec((TB, HIDDEN), lambda i: (i, 0)),
            pl.BlockSpec((HIDDEN, E), lambda i: (0, 0)),
            pl.BlockSpec((1, HIDDEN), lambda i: (0, 0)),
            pl.BlockSpec((1, E), lambda i: (0, 0)),
        ],
        out_specs=(colspec, colspec, colspec, colspec, colspec, colspec,
                   pl.BlockSpec((1, E), lambda i: (0, 0))),
        out_shape=out_shape,
        scratch_shapes=[pltpu.VMEM((1, E), jnp.float32)],
    )(x, wt, scale2d, pes2d)


def _scatter_body(s0, s1, e0, e1, r0, r1, cnt, out_s, out_t,
                  sh_s, sh_t, cnt_v, off_v, e_v, r_v, val_v, tok_v, dest_v,
                  stage_s, stage_t, sem):
    wid = lax.axis_index("s")
    base = wid * SCHUNK

    pltpu.sync_copy(cnt, cnt_v)

    carry = jnp.int32(0)
    for j in range(E // 16):
        v = cnt_v[pl.ds(j * 16, 16)]
        c = plsc.cumsum(v)
        off_v[pl.ds(j * 16, 16)] = (c - v) + carry
        carry = carry + jnp.sum(v)

    for k in range(SCHUNK // 16):
        tok_v[pl.ds(k * 16, 16)] = lax.iota(jnp.int32, 16) + (base + k * 16)

    copies = []
    for col in range(2):
        e_hbm = (e0, e1)[col]
        r_hbm = (r0, r1)[col]
        s_hbm = (s0, s1)[col]
        pltpu.sync_copy(e_hbm.at[pl.ds(base, SCHUNK)], e_v)
        pltpu.sync_copy(r_hbm.at[pl.ds(base, SCHUNK)], r_v)
        pltpu.sync_copy(s_hbm.at[pl.ds(base, SCHUNK)], val_v)
        for j in range(SCHUNK // 128):
            for i in range(8):
                k = j * 8 + i
                ev = e_v[pl.ds(k * 16, 16)]
                rv = r_v[pl.ds(k * 16, 16)]
                dest_v[col, j, pl.ds(i * 16, 16)] = plsc.load_gather(off_v, [ev]) + rv
        for j in range(SCHUNK // 128):
            src = pl.ds(j * 128, 128)
            copies.append(pltpu.async_copy(val_v.at[src], sh_s.at[dest_v.at[col, j]], sem))
            copies.append(pltpu.async_copy(tok_v.at[src], sh_t.at[dest_v.at[col, j]], sem))
        if col == 0:
            for cp in copies:
                cp.wait()
            copies = []
    for cp in copies:
        cp.wait()
    plsc.subcore_barrier()
    out_slice = pl.ds(wid * (2 * TOK // NS), 2 * TOK // NS)
    pltpu.sync_copy(sh_s.at[out_slice], stage_s)
    pltpu.sync_copy(stage_s, out_s.at[out_slice])
    pltpu.sync_copy(sh_t.at[out_slice], stage_t)
    pltpu.sync_copy(stage_t, out_t.at[out_slice])


def _stage2(s0, s1, e0, e1, r0, r1, cnt):
    mesh = plsc.VectorSubcoreMesh(core_axis_name="c", subcore_axis_name="s",
                                  num_cores=1, num_subcores=NS)
    run = pl.kernel(
        _scatter_body,
        out_type=(jax.ShapeDtypeStruct((2 * TOK,), jnp.float32),
                  jax.ShapeDtypeStruct((2 * TOK,), jnp.int32)),
        mesh=mesh,
        scratch_types=[
            pltpu.VMEM_SHARED((2 * TOK,), jnp.float32),
            pltpu.VMEM_SHARED((2 * TOK,), jnp.int32),
            pltpu.VMEM((E,), jnp.int32),
            pltpu.VMEM((E,), jnp.int32),
            pltpu.VMEM((SCHUNK,), jnp.int32),
            pltpu.VMEM((SCHUNK,), jnp.int32),
            pltpu.VMEM((SCHUNK,), jnp.float32),
            pltpu.VMEM((SCHUNK,), jnp.int32),
            pltpu.VMEM((2, SCHUNK // 128, 128), jnp.int32),
            pltpu.VMEM((2 * TOK // NS,), jnp.float32),
            pltpu.VMEM((2 * TOK // NS,), jnp.int32),
            pltpu.SemaphoreType.DMA,
        ],
        compiler_params=pltpu.CompilerParams(needs_layout_passes=False),
    )
    return run(s0, s1, e0, e1, r0, r1, cnt)


def kernel(x, W, scale, per_expert_scale):
    wt = W.T
    scale2d = scale.reshape(1, HIDDEN)
    pes2d = per_expert_scale.reshape(1, E)
    s0, s1, e0, e1, r0, r1, cnt = _stage1(x, wt, scale2d, pes2d)
    out_s, out_t = _stage2(s0.reshape(TOK), s1.reshape(TOK),
                           e0.reshape(TOK), e1.reshape(TOK),
                           r0.reshape(TOK), r1.reshape(TOK),
                           cnt.reshape(E))
    return out_s, out_t, cnt.reshape(E)


# speedup vs baseline: 2.1817x; 1.0555x over previous
import functools

import jax
import jax.numpy as jnp
from jax import lax
from jax.experimental import pallas as pl
from jax.experimental.pallas import tpu as pltpu
from jax.experimental.pallas import tpu_sc as plsc

HIDDEN = 4096
E = 64
TOK = 16384
EPS = 1e-06

TB = 1024
NBLK = TOK // TB
NC, NS = 2, 16
SCHUNK = TOK // NS


def _router_block(x_ref, wt_ref, scale_ref, pes_ref,
                  s0_ref, s1_ref, e0_ref, e1_ref, r0_ref, r1_ref, cnt_ref,
                  run_ref):
    i = pl.program_id(0)

    @pl.when(i == 0)
    def _():
        run_ref[...] = jnp.zeros_like(run_ref)

    xf = x_ref[...]
    ms = jnp.mean(xf * xf, axis=-1, keepdims=True)
    normed = (xf * lax.rsqrt(ms + EPS)) * scale_ref[...]
    logits = jnp.dot(normed, wt_ref[...], preferred_element_type=jnp.float32)
    scores = jax.nn.sigmoid(logits * pes_ref[...])

    iota = lax.broadcasted_iota(jnp.int32, (TB, E), 1)
    m1 = jnp.max(scores, axis=1, keepdims=True)
    i1 = jnp.min(jnp.where(scores == m1, iota, E), axis=1, keepdims=True)
    masked = jnp.where(iota == i1, -1.0, scores)
    m2 = jnp.max(masked, axis=1, keepdims=True)
    i2 = jnp.min(jnp.where(masked == m2, iota, E), axis=1, keepdims=True)

    oh0 = (iota == i1).astype(jnp.float32)
    oh1 = (iota == i2).astype(jnp.float32)
    ohs = oh0 + oh1

    ri = lax.broadcasted_iota(jnp.int32, (TB, TB), 0)
    ci = lax.broadcasted_iota(jnp.int32, (TB, TB), 1)
    ltri = (ci < ri).astype(jnp.float32)
    S = jnp.dot(ltri, ohs, preferred_element_type=jnp.float32)

    srun = S + run_ref[...]
    r0 = jnp.sum(srun * oh0, axis=1, keepdims=True)
    r1 = jnp.sum(srun * oh1, axis=1, keepdims=True)

    s0_ref[...] = m1
    s1_ref[...] = m2
    e0_ref[...] = i1
    e1_ref[...] = i2
    r0_ref[...] = r0.astype(jnp.int32)
    r1_ref[...] = r1.astype(jnp.int32)

    run_new = run_ref[...] + jnp.sum(ohs, axis=0, keepdims=True)
    run_ref[...] = run_new
    cnt_ref[...] = run_new.astype(jnp.int32)


def _stage1(x, wt, scale2d, pes2d):
    grid = (NBLK,)
    col = lambda dt: jax.ShapeDtypeStruct((TOK, 1), dt)
    out_shape = (col(jnp.float32), col(jnp.float32),
                 col(jnp.int32), col(jnp.int32),
                 col(jnp.int32), col(jnp.int32),
                 jax.ShapeDtypeStruct((1, E), jnp.int32))
    colspec = pl.BlockSpec((TB, 1), lambda i: (i, 0))
    return pl.pallas_call(
        _router_block,
        grid=grid,
        in_specs=[
            pl.BlockSpec((TB, HIDDEN), lambda i: (i, 0)),
            pl.BlockSpec((HIDDEN, E), lambda i: (0, 0)),
            pl.BlockSpec((1, HIDDEN), lambda i: (0, 0)),
            pl.BlockSpec((1, E), lambda i: (0, 0)),
        ],
        out_specs=(colspec, colspec, colspec, colspec, colspec, colspec,
                   pl.BlockSpec((1, E), lambda i: (0, 0))),
        out_shape=out_shape,
        scratch_shapes=[pltpu.VMEM((1, E), jnp.float32)],
    )(x, wt, scale2d, pes2d)


def _scatter_body(s0, s1, e0, e1, r0, r1, cnt, out_s, out_t,
                  sh_s, sh_t, cnt_v, off_v, e_v, r_v, val_v, tok_v, dest_v,
                  stage_s, stage_t, sem):
    wid = lax.axis_index("s")
    base = wid * SCHUNK

    pltpu.sync_copy(cnt, cnt_v)

    carry = jnp.int32(0)
    for j in range(E // 16):
        v = cnt_v[pl.ds(j * 16, 16)]
        c = plsc.cumsum(v)
        off_v[pl.ds(j * 16, 16)] = (c - v) + carry
        carry = carry + jnp.sum(v)

    for k in range(SCHUNK // 16):
        tok_v[pl.ds(k * 16, 16)] = lax.iota(jnp.int32, 16) + (base + k * 16)

    copies = []
    for col in range(2):
        e_hbm = (e0, e1)[col]
        r_hbm = (r0, r1)[col]
        s_hbm = (s0, s1)[col]
        pltpu.sync_copy(e_hbm.at[pl.ds(base, SCHUNK)], e_v)
        pltpu.sync_copy(r_hbm.at[pl.ds(base, SCHUNK)], r_v)
        pltpu.sync_copy(s_hbm.at[pl.ds(base, SCHUNK)], val_v)
        for j in range(SCHUNK // 128):
            for i in range(8):
                k = j * 8 + i
                ev = e_v[pl.ds(k * 16, 16)]
                rv = r_v[pl.ds(k * 16, 16)]
                dest_v[col, j, pl.ds(i * 16, 16)] = plsc.load_gather(off_v, [ev]) + rv
        for j in range(SCHUNK // 128):
            src = pl.ds(j * 128, 128)
            copies.append(pltpu.async_copy(val_v.at[src], sh_s.at[dest_v.at[col, j]], sem))
            copies.append(pltpu.async_copy(tok_v.at[src], sh_t.at[dest_v.at[col, j]], sem))
        if col == 0:
            for cp in copies:
                cp.wait()
            copies = []
    for cp in copies:
        cp.wait()
    plsc.subcore_barrier()
    out_slice = pl.ds(wid * (2 * TOK // NS), 2 * TOK // NS)
    pltpu.sync_copy(sh_s.at[out_slice], stage_s)
    pltpu.sync_copy(stage_s, out_s.at[out_slice])
    pltpu.sync_copy(sh_t.at[out_slice], stage_t)
    pltpu.sync_copy(stage_t, out_t.at[out_slice])


def _stage2(s0, s1, e0, e1, r0, r1, cnt):
    mesh = plsc.VectorSubcoreMesh(core_axis_name="c", subcore_axis_name="s",
                                  num_cores=1, num_subcores=NS)
    run = pl.kernel(
        _scatter_body,
        out_type=(jax.ShapeDtypeStruct((2 * TOK,), jnp.float32),
                  jax.ShapeDtypeStruct((2 * TOK,), jnp.int32)),
        mesh=mesh,
        scratch_types=[
            pltpu.VMEM_SHARED((2 * TOK,), jnp.float32),
            pltpu.VMEM_SHARED((2 * TOK,), jnp.int32),
            pltpu.VMEM((E,), jnp.int32),
            pltpu.VMEM((E,), jnp.int32),
            pltpu.VMEM((SCHUNK,), jnp.int32),
            pltpu.VMEM((SCHUNK,), jnp.int32),
            pltpu.VMEM((SCHUNK,), jnp.float32),
            pltpu.VMEM((SCHUNK,), jnp.int32),
            pltpu.VMEM((2, SCHUNK // 128, 128), jnp.int32),
            pltpu.VMEM((2 * TOK // NS,), jnp.float32),
            pltpu.VMEM((2 * TOK // NS,), jnp.int32),
            pltpu.SemaphoreType.DMA,
        ],
        compiler_params=pltpu.CompilerParams(needs_layout_passes=False),
    )
    return run(s0, s1, e0, e1, r0, r1, cnt)


def kernel(x, W, scale, per_expert_scale):
    wt = W.T
    scale2d = scale.reshape(1, HIDDEN)
    pes2d = per_expert_scale.reshape(1, E)
    s0, s1, e0, e1, r0, r1, cnt = _stage1(x, wt, scale2d, pes2d)
    out_s, out_t = _stage2(s0.reshape(TOK), s1.reshape(TOK),
                           e0.reshape(TOK), e1.reshape(TOK),
                           r0.reshape(TOK), r1.reshape(TOK),
                           cnt.reshape(E))
    return out_s, out_t, cnt.reshape(E)
